# Initial kernel scaffold; baseline (speedup 1.0000x reference)
import functools

import jax
import jax.numpy as jnp
from jax import lax
from jax.experimental import pallas as pl
from jax.experimental.pallas import tpu as pltpu
from jax.experimental.pallas import tpu_sc as plsc

N = 8192
D = 512
BM = 512
BN = 512
NI = N // BM
NJ = N // BN

_INT_BIG = jnp.int32(2**30)


def _argmin_body(a_ref, b_ref, pred_ref, best_val, best_idx):
    i = pl.program_id(0)
    j = pl.program_id(1)

    @pl.when(j == 0)
    def _init():
        best_val[...] = jnp.full((BM, 1), jnp.inf, jnp.float32)
        best_idx[...] = jnp.zeros((BM, 1), jnp.int32)

    a = a_ref[...]
    b = b_ref[...]
    g = lax.dot_general(a, b, (((1,), (1,)), ((), ())),
                        preferred_element_type=jnp.float32)
    nb = jnp.sum(b * b, axis=1).reshape(1, BN)
    d = nb - (g + g)

    col = lax.broadcasted_iota(jnp.int32, (BM, BN), 1) + j * BN
    row = lax.broadcasted_iota(jnp.int32, (BM, BN), 0) + i * BM
    d = jnp.where((i == j) & (row == col), jnp.inf, d)

    m = jnp.min(d, axis=1, keepdims=True)
    idx = jnp.min(jnp.where(d == m, col, _INT_BIG), axis=1, keepdims=True)

    upd = m < best_val[...]
    best_idx[...] = jnp.where(upd, idx, best_idx[...])
    best_val[...] = jnp.where(upd, m, best_val[...])

    @pl.when(j == NJ - 1)
    def _emit():
        pred_ref[0, 0, :] = best_idx[...].reshape(BM)


def _nearest_neighbor(feature_bank):
    pred3 = pl.pallas_call(
        _argmin_body,
        grid=(NI, NJ),
        in_specs=[
            pl.BlockSpec((BM, D), lambda i, j: (i, 0)),
            pl.BlockSpec((BN, D), lambda i, j: (j, 0)),
        ],
        out_specs=pl.BlockSpec((1, 1, BM), lambda i, j: (i, 0, 0)),
        out_shape=jax.ShapeDtypeStruct((NI, 1, BM), jnp.int32),
        scratch_shapes=[
            pltpu.VMEM((BM, 1), jnp.float32),
            pltpu.VMEM((BM, 1), jnp.int32),
        ],
    )(feature_bank, feature_bank)
    return pred3.reshape(N)


_SC_INFO = plsc.get_sparse_core_info()
_NW = _SC_INFO.num_cores * _SC_INFO.num_subcores
_CHUNK = N // _NW
_L = _SC_INFO.num_lanes


def _sc_match_counts(pred, label_bank):
    mesh = plsc.VectorSubcoreMesh(core_axis_name="c", subcore_axis_name="s")

    @functools.partial(
        pl.kernel,
        mesh=mesh,
        out_type=jax.ShapeDtypeStruct((_NW, _L), jnp.int32),
        scratch_types=[
            pltpu.VMEM((N,), jnp.int32),
            pltpu.VMEM((_CHUNK,), jnp.int32),
            pltpu.VMEM((_CHUNK,), jnp.int32),
            pltpu.VMEM((_L,), jnp.int32),
        ],
    )
    def k(pred_hbm, label_hbm, out_hbm, table_v, idx_v, own_v, acc_v):
        wid = lax.axis_index("s") * _SC_INFO.num_cores + lax.axis_index("c")
        base = wid * _CHUNK
        pltpu.sync_copy(label_hbm, table_v)
        pltpu.sync_copy(pred_hbm.at[pl.ds(base, _CHUNK)], idx_v)
        pltpu.sync_copy(label_hbm.at[pl.ds(base, _CHUNK)], own_v)
        acc = jnp.zeros((_L,), jnp.int32)
        for t in range(_CHUNK // _L):
            idx = idx_v[pl.ds(t * _L, _L)]
            own = own_v[pl.ds(t * _L, _L)]
            g = plsc.load_gather(table_v, [idx])
            acc = acc + jnp.where(g == own, 1, 0).astype(jnp.int32)
        acc_v[...] = acc
        pltpu.sync_copy(acc_v, out_hbm.at[wid])

    return k(pred, label_bank)


def kernel(feature_bank, label_bank):
    pred = _nearest_neighbor(feature_bank)
    counts = _sc_match_counts(pred, label_bank)
    return jnp.sum(counts).astype(jnp.float32) / jnp.float32(N)


# speedup vs baseline: 4.9057x; 4.9057x over previous
import functools

import jax
import jax.numpy as jnp
from jax import lax
from jax.experimental import pallas as pl
from jax.experimental.pallas import tpu as pltpu
from jax.experimental.pallas import tpu_sc as plsc

N = 8192
D = 512
BM = 512
BN = 512
NI = N // BM
NJ = N // BN

def _argmin_body(a_ref, b_ref, pred_ref, best_val, best_idx):
    i = pl.program_id(0)
    j = pl.program_id(1)

    @pl.when(j == 0)
    def _init():
        best_val[...] = jnp.full((BM, 1), jnp.inf, jnp.float32)
        best_idx[...] = jnp.zeros((BM, 1), jnp.int32)

    a = a_ref[...]
    b = b_ref[...]
    g = lax.dot_general(a, b, (((1,), (1,)), ((), ())),
                        preferred_element_type=jnp.float32)
    nb8 = lax.dot_general(jnp.ones((8, D), jnp.float32), b * b,
                          (((1,), (1,)), ((), ())),
                          preferred_element_type=jnp.float32)
    nb = nb8[0:1, :]
    d = nb - (g + g)

    col = lax.broadcasted_iota(jnp.int32, (BM, BN), 1) + j * BN
    row = lax.broadcasted_iota(jnp.int32, (BM, BN), 0) + i * BM
    d = jnp.where(row == col, jnp.inf, d)

    m = jnp.min(d, axis=1, keepdims=True)
    idx = jnp.min(jnp.where(d == m, col, jnp.int32(2**30)), axis=1, keepdims=True)

    upd = m < best_val[...]
    best_idx[...] = jnp.where(upd, idx, best_idx[...])
    best_val[...] = jnp.where(upd, m, best_val[...])

    @pl.when(j == NJ - 1)
    def _emit():
        pred_ref[0, 0, :] = best_idx[...].reshape(BM)


def _nearest_neighbor(feature_bank):
    pred3 = pl.pallas_call(
        _argmin_body,
        grid=(NI, NJ),
        in_specs=[
            pl.BlockSpec((BM, D), lambda i, j: (i, 0)),
            pl.BlockSpec((BN, D), lambda i, j: (j, 0)),
        ],
        out_specs=pl.BlockSpec((1, 1, BM), lambda i, j: (i, 0, 0)),
        out_shape=jax.ShapeDtypeStruct((NI, 1, BM), jnp.int32),
        scratch_shapes=[
            pltpu.VMEM((BM, 1), jnp.float32),
            pltpu.VMEM((BM, 1), jnp.int32),
        ],
    )(feature_bank, feature_bank)
    return pred3.reshape(N)


_NC = 2
_NS = 16
_NW = _NC * _NS
_CHUNK = N // _NW
_L = 16


def _sc_match_counts(pred, label_bank):
    mesh = plsc.VectorSubcoreMesh(core_axis_name="c", subcore_axis_name="s")

    @functools.partial(
        pl.kernel,
        mesh=mesh,
        out_type=jax.ShapeDtypeStruct((_NW, _L), jnp.int32),
        scratch_types=[
            pltpu.VMEM((N,), jnp.int32),
            pltpu.VMEM((_CHUNK,), jnp.int32),
            pltpu.VMEM((_CHUNK,), jnp.int32),
            pltpu.VMEM((_L,), jnp.int32),
        ],
        compiler_params=pltpu.CompilerParams(needs_layout_passes=False),
    )
    def k(pred_hbm, label_hbm, out_hbm, table_v, idx_v, own_v, acc_v):
        wid = lax.axis_index("s") * _NC + lax.axis_index("c")
        base = wid * _CHUNK
        pltpu.sync_copy(label_hbm, table_v)
        pltpu.sync_copy(pred_hbm.at[pl.ds(base, _CHUNK)], idx_v)
        pltpu.sync_copy(label_hbm.at[pl.ds(base, _CHUNK)], own_v)
        acc = jnp.zeros((_L,), jnp.int32)
        for t in range(_CHUNK // _L):
            idx = idx_v[pl.ds(t * _L, _L)]
            own = own_v[pl.ds(t * _L, _L)]
            g = plsc.load_gather(table_v, [idx])
            acc = acc + jnp.where(g == own, 1, 0).astype(jnp.int32)
        acc_v[...] = acc
        pltpu.sync_copy(acc_v, out_hbm.at[wid])

    return k(pred, label_bank)


def kernel(feature_bank, label_bank):
    pred = _nearest_neighbor(feature_bank)
    counts = _sc_match_counts(pred, label_bank)
    return jnp.sum(counts).astype(jnp.float32) / jnp.float32(N)


# speedup vs baseline: 6.6294x; 1.3514x over previous
import functools

import jax
import jax.numpy as jnp
from jax import lax
from jax.experimental import pallas as pl
from jax.experimental.pallas import tpu as pltpu
from jax.experimental.pallas import tpu_sc as plsc

N = 8192
D = 512
BM = 512
BN = 512
NI = N // BM
NJ = N // BN
NJJ = NI // 2 + 1

def _argmin_body(a_ref, b_ref, rv_ref, ri_ref, cv_ref, ci_ref):
    i = pl.program_id(0)
    jj = pl.program_id(1)
    j = lax.rem(i + jj, NI)

    @pl.when((i == 0) & (jj == 0))
    def _init():
        rv_ref[...] = jnp.full((BM, NI), jnp.inf, jnp.float32)
        ri_ref[...] = jnp.zeros((BM, NI), jnp.int32)
        cv_ref[...] = jnp.full((NI, BN), jnp.inf, jnp.float32)
        ci_ref[...] = jnp.zeros((NI, BN), jnp.int32)

    @pl.when((jj != NI // 2) | (i < NI // 2))
    def _compute():
        a = a_ref[...]
        b = b_ref[...]
        g = lax.dot_general(a, b, (((1,), (1,)), ((), ())),
                            preferred_element_type=jnp.float32)
        u = g + g
        nb8 = lax.dot_general(jnp.ones((8, D), jnp.float32), b * b,
                              (((1,), (1,)), ((), ())),
                              preferred_element_type=jnp.float32)
        nb = nb8[0:1, :]

        lrow = lax.broadcasted_iota(jnp.int32, (BM, BN), 0)
        lcol = lax.broadcasted_iota(jnp.int32, (BM, BN), 1)

        d = nb - u
        d = jnp.where((jj == 0) & (lrow == lcol), jnp.inf, d)
        m = jnp.min(d, axis=1, keepdims=True)
        idx = jnp.min(jnp.where(d == m, lcol, jnp.int32(2**30)), axis=1,
                      keepdims=True) + j * BN
        bv = rv_ref[...]
        bi = ri_ref[...]
        colmask = lax.broadcasted_iota(jnp.int32, (BM, NI), 1) == i
        upd = colmask & ((m < bv) | ((m == bv) & (idx < bi)))
        rv_ref[...] = jnp.where(upd, m, bv)
        ri_ref[...] = jnp.where(upd, idx, bi)

        @pl.when(jj != 0)
        def _col_side():
            na = jnp.sum(a * a, axis=1, keepdims=True)
            dc = na - u
            mc = jnp.min(dc, axis=0, keepdims=True)
            idc = jnp.min(jnp.where(dc == mc, lrow, jnp.int32(2**30)), axis=0,
                          keepdims=True) + i * BM
            cbv = cv_ref[...]
            cbi = ci_ref[...]
            rowmask = lax.broadcasted_iota(jnp.int32, (NI, BN), 0) == j
            cupd = rowmask & ((mc < cbv) | ((mc == cbv) & (idc < cbi)))
            cv_ref[...] = jnp.where(cupd, mc, cbv)
            ci_ref[...] = jnp.where(cupd, idc, cbi)


def _nearest_neighbor_halves(feature_bank):
    rv, ri, cv, ci = pl.pallas_call(
        _argmin_body,
        grid=(NI, NJJ),
        in_specs=[
            pl.BlockSpec((BM, D), lambda i, jj: (i, 0)),
            pl.BlockSpec((BN, D), lambda i, jj: ((i + jj) % NI, 0)),
        ],
        out_specs=[
            pl.BlockSpec((BM, NI), lambda i, jj: (0, 0)),
            pl.BlockSpec((BM, NI), lambda i, jj: (0, 0)),
            pl.BlockSpec((NI, BN), lambda i, jj: (0, 0)),
            pl.BlockSpec((NI, BN), lambda i, jj: (0, 0)),
        ],
        out_shape=[
            jax.ShapeDtypeStruct((BM, NI), jnp.float32),
            jax.ShapeDtypeStruct((BM, NI), jnp.int32),
            jax.ShapeDtypeStruct((NI, BN), jnp.float32),
            jax.ShapeDtypeStruct((NI, BN), jnp.int32),
        ],
    )(feature_bank, feature_bank)
    return (rv.T.reshape(N), ri.T.reshape(N),
            cv.reshape(N), ci.reshape(N))


_NC = 2
_NS = 16
_NW = _NC * _NS
_CHUNK = N // _NW
_L = 16


def _sc_merge_and_count(rv, ri, cv, ci, label_bank):
    mesh = plsc.VectorSubcoreMesh(core_axis_name="c", subcore_axis_name="s")

    @functools.partial(
        pl.kernel,
        mesh=mesh,
        out_type=jax.ShapeDtypeStruct((_NW, _L), jnp.int32),
        scratch_types=[
            pltpu.VMEM((N,), jnp.int32),
            pltpu.VMEM((_CHUNK,), jnp.float32),
            pltpu.VMEM((_CHUNK,), jnp.int32),
            pltpu.VMEM((_CHUNK,), jnp.float32),
            pltpu.VMEM((_CHUNK,), jnp.int32),
            pltpu.VMEM((_CHUNK,), jnp.int32),
            pltpu.VMEM((_L,), jnp.int32),
        ],
        compiler_params=pltpu.CompilerParams(needs_layout_passes=False),
    )
    def k(rv_hbm, ri_hbm, cv_hbm, ci_hbm, label_hbm, out_hbm,
          table_v, rv_v, ri_v, cv_v, ci_v, own_v, acc_v):
        wid = lax.axis_index("s") * _NC + lax.axis_index("c")
        base = wid * _CHUNK
        pltpu.sync_copy(label_hbm, table_v)
        pltpu.sync_copy(rv_hbm.at[pl.ds(base, _CHUNK)], rv_v)
        pltpu.sync_copy(ri_hbm.at[pl.ds(base, _CHUNK)], ri_v)
        pltpu.sync_copy(cv_hbm.at[pl.ds(base, _CHUNK)], cv_v)
        pltpu.sync_copy(ci_hbm.at[pl.ds(base, _CHUNK)], ci_v)
        pltpu.sync_copy(label_hbm.at[pl.ds(base, _CHUNK)], own_v)
        acc = jnp.zeros((_L,), jnp.int32)
        for t in range(_CHUNK // _L):
            s = pl.ds(t * _L, _L)
            rvv, riv = rv_v[s], ri_v[s]
            cvv, civ = cv_v[s], ci_v[s]
            own = own_v[s]
            sel = (cvv < rvv) | ((cvv == rvv) & (civ < riv))
            pred = jnp.where(sel, civ, riv)
            g = plsc.load_gather(table_v, [pred])
            acc = acc + jnp.where(g == own, 1, 0).astype(jnp.int32)
        acc_v[...] = acc
        pltpu.sync_copy(acc_v, out_hbm.at[wid])

    return k(rv, ri, cv, ci, label_bank)


def kernel(feature_bank, label_bank):
    rv, ri, cv, ci = _nearest_neighbor_halves(feature_bank)
    counts = _sc_merge_and_count(rv, ri, cv, ci, label_bank)
    return jnp.sum(counts).astype(jnp.float32) / jnp.float32(N)


# speedup vs baseline: 7.1281x; 1.0752x over previous
import functools

import jax
import jax.numpy as jnp
from jax import lax
from jax.experimental import pallas as pl
from jax.experimental.pallas import tpu as pltpu
from jax.experimental.pallas import tpu_sc as plsc

N = 8192
D = 512
BM = 512
BN = 512
NI = N // BM
NJJ = NI // 2 + 1
NT = NI * NJJ


def _coords(t):
    i = t // NJJ
    jj = lax.rem(t, NJJ)
    j = lax.rem(i + jj, NI)
    return i, jj, j


def _argmin_body(bank_ref, rv_ref, ri_ref, cv_ref, ci_ref,
                 gbuf, nrow_ref, ncol_ref):
    s = pl.program_id(0)

    @pl.when(s == 0)
    def _prologue():
        rv_ref[...] = jnp.full((BM, NI), jnp.inf, jnp.float32)
        ri_ref[...] = jnp.zeros((BM, NI), jnp.int32)
        cv_ref[...] = jnp.full((NI, BN), jnp.inf, jnp.float32)
        ci_ref[...] = jnp.zeros((NI, BN), jnp.int32)
        lane16 = lax.broadcasted_iota(jnp.int32, (BM, NI), 1)
        for blk in range(NI):
            bs = bank_ref[blk * BM:(blk + 1) * BM, :]
            sq = bs * bs
            nb8 = lax.dot_general(jnp.ones((8, D), jnp.float32), sq,
                                  (((1,), (1,)), ((), ())),
                                  preferred_element_type=jnp.float32)
            nrow_ref[pl.ds(blk, 1), :, :] = nb8.reshape(1, 8, BN)
            na = jnp.sum(sq, axis=1, keepdims=True)
            ncol_ref[...] = jnp.where(lane16 == blk, na, ncol_ref[...])

    tp = jnp.maximum(s - 1, 0)
    i_p, jj_p, j_p = _coords(tp)

    @pl.when((s >= 1) & ((jj_p != NI // 2) | (i_p < NI // 2)))
    def _process():
        g = gbuf[...]
        u = g + g
        nb = nrow_ref[pl.ds(j_p, 1), :, :].reshape(8, BN)[0:1, :]
        lrow = lax.broadcasted_iota(jnp.int32, (BM, BN), 0)
        lcol = lax.broadcasted_iota(jnp.int32, (BM, BN), 1)

        d = nb - u
        d = jnp.where((jj_p == 0) & (lrow == lcol), jnp.inf, d)
        m = jnp.min(d, axis=1, keepdims=True)
        idx = jnp.min(jnp.where(d == m, lcol, jnp.int32(2**30)), axis=1,
                      keepdims=True) + j_p * BN
        bv = rv_ref[...]
        bi = ri_ref[...]
        colmask = lax.broadcasted_iota(jnp.int32, (BM, NI), 1) == i_p
        upd = colmask & ((m < bv) | ((m == bv) & (idx < bi)))
        rv_ref[...] = jnp.where(upd, m, bv)
        ri_ref[...] = jnp.where(upd, idx, bi)

        @pl.when(jj_p != 0)
        def _col_side():
            nacol = jnp.sum(
                jnp.where(colmask, ncol_ref[...], 0.0), axis=1,
                keepdims=True)
            dc = nacol - u
            mc = jnp.min(dc, axis=0, keepdims=True)
            idc = jnp.min(jnp.where(dc == mc, lrow, jnp.int32(2**30)),
                          axis=0, keepdims=True) + i_p * BM
            cbv = cv_ref[...]
            cbi = ci_ref[...]
            rowmask = lax.broadcasted_iota(jnp.int32, (NI, BN), 0) == j_p
            cupd = rowmask & ((mc < cbv) | ((mc == cbv) & (idc < cbi)))
            cv_ref[...] = jnp.where(cupd, mc, cbv)
            ci_ref[...] = jnp.where(cupd, idc, cbi)

    i_c, jj_c, j_c = _coords(jnp.minimum(s, NT - 1))

    @pl.when((s < NT) & ((jj_c != NI // 2) | (i_c < NI // 2)))
    def _compute():
        a = bank_ref[pl.ds(pl.multiple_of(i_c * BM, BM), BM), :]
        b = bank_ref[pl.ds(pl.multiple_of(j_c * BN, BN), BN), :]
        gbuf[...] = lax.dot_general(a, b, (((1,), (1,)), ((), ())),
                                    preferred_element_type=jnp.float32)


def _nearest_neighbor_halves(feature_bank):
    rv, ri, cv, ci = pl.pallas_call(
        _argmin_body,
        grid=(NT + 1,),
        in_specs=[pl.BlockSpec((N, D), lambda s: (0, 0))],
        out_specs=[
            pl.BlockSpec((BM, NI), lambda s: (0, 0)),
            pl.BlockSpec((BM, NI), lambda s: (0, 0)),
            pl.BlockSpec((NI, BN), lambda s: (0, 0)),
            pl.BlockSpec((NI, BN), lambda s: (0, 0)),
        ],
        out_shape=[
            jax.ShapeDtypeStruct((BM, NI), jnp.float32),
            jax.ShapeDtypeStruct((BM, NI), jnp.int32),
            jax.ShapeDtypeStruct((NI, BN), jnp.float32),
            jax.ShapeDtypeStruct((NI, BN), jnp.int32),
        ],
        scratch_shapes=[
            pltpu.VMEM((BM, BN), jnp.float32),
            pltpu.VMEM((NI, 8, BN), jnp.float32),
            pltpu.VMEM((BM, NI), jnp.float32),
        ],
    )(feature_bank)
    return (rv.T.reshape(N), ri.T.reshape(N),
            cv.reshape(N), ci.reshape(N))


_NC = 2
_NS = 16
_NW = _NC * _NS
_CHUNK = N // _NW
_L = 16


def _sc_merge_and_count(rv, ri, cv, ci, label_bank):
    mesh = plsc.VectorSubcoreMesh(core_axis_name="c", subcore_axis_name="s")

    @functools.partial(
        pl.kernel,
        mesh=mesh,
        out_type=jax.ShapeDtypeStruct((_NW, _L), jnp.int32),
        scratch_types=[
            pltpu.VMEM((N,), jnp.int32),
            pltpu.VMEM((_CHUNK,), jnp.float32),
            pltpu.VMEM((_CHUNK,), jnp.int32),
            pltpu.VMEM((_CHUNK,), jnp.float32),
            pltpu.VMEM((_CHUNK,), jnp.int32),
            pltpu.VMEM((_CHUNK,), jnp.int32),
            pltpu.VMEM((_L,), jnp.int32),
        ],
        compiler_params=pltpu.CompilerParams(needs_layout_passes=False),
    )
    def k(rv_hbm, ri_hbm, cv_hbm, ci_hbm, label_hbm, out_hbm,
          table_v, rv_v, ri_v, cv_v, ci_v, own_v, acc_v):
        wid = lax.axis_index("s") * _NC + lax.axis_index("c")
        base = wid * _CHUNK
        pltpu.sync_copy(label_hbm, table_v)
        pltpu.sync_copy(rv_hbm.at[pl.ds(base, _CHUNK)], rv_v)
        pltpu.sync_copy(ri_hbm.at[pl.ds(base, _CHUNK)], ri_v)
        pltpu.sync_copy(cv_hbm.at[pl.ds(base, _CHUNK)], cv_v)
        pltpu.sync_copy(ci_hbm.at[pl.ds(base, _CHUNK)], ci_v)
        pltpu.sync_copy(label_hbm.at[pl.ds(base, _CHUNK)], own_v)
        acc = jnp.zeros((_L,), jnp.int32)
        for t in range(_CHUNK // _L):
            sl = pl.ds(t * _L, _L)
            rvv, riv = rv_v[sl], ri_v[sl]
            cvv, civ = cv_v[sl], ci_v[sl]
            own = own_v[sl]
            sel = (cvv < rvv) | ((cvv == rvv) & (civ < riv))
            pred = jnp.where(sel, civ, riv)
            g = plsc.load_gather(table_v, [pred])
            acc = acc + jnp.where(g == own, 1, 0).astype(jnp.int32)
        acc_v[...] = acc
        pltpu.sync_copy(acc_v, out_hbm.at[wid])

    return k(rv, ri, cv, ci, label_bank)


def kernel(feature_bank, label_bank):
    rv, ri, cv, ci = _nearest_neighbor_halves(feature_bank)
    counts = _sc_merge_and_count(rv, ri, cv, ci, label_bank)
    return jnp.sum(counts).astype(jnp.float32) / jnp.float32(N)


# speedup vs baseline: 9.4797x; 1.3299x over previous
import functools

import jax
import jax.numpy as jnp
from jax import lax
from jax.experimental import pallas as pl
from jax.experimental.pallas import tpu as pltpu
from jax.experimental.pallas import tpu_sc as plsc

N = 8192
D = 512
BM = 512
BN = 512
NI = N // BM
NJJ = NI // 2 + 1
NT = NI * NJJ


def _coords(t):
    i = t // NJJ
    jj = lax.rem(t, NJJ)
    j = lax.rem(i + jj, NI)
    return i, jj, j


def _argmin_body(bank_ref, rv_ref, ri_ref, cv_ref, ci_ref,
                 gbuf, nrow_ref, napg_ref, bank2_ref, bias_ref,
                 revc_ref, revr_ref):
    s = pl.program_id(0)

    @pl.when(s == 0)
    def _prologue():
        rv_ref[...] = jnp.full((BM, NI), jnp.inf, jnp.float32)
        ri_ref[...] = jnp.zeros((BM, NI), jnp.int32)
        cv_ref[...] = jnp.full((NI, BN), jnp.inf, jnp.float32)
        ci_ref[...] = jnp.zeros((NI, BN), jnp.int32)
        gbuf[...] = jnp.zeros((BM, BN), jnp.float32)
        bank = bank_ref[...]
        bank2_ref[...] = bank + bank
        lrow = lax.broadcasted_iota(jnp.int32, (BM, BN), 0)
        lcol = lax.broadcasted_iota(jnp.int32, (BM, BN), 1)
        bias_ref[pl.ds(0, 1), :, :] = jnp.zeros((1, BM, BN), jnp.float32)
        bias_ref[pl.ds(1, 1), :, :] = jnp.where(
            lrow == lcol, -jnp.inf, 0.0).reshape(1, BM, BN)
        revc_ref[...] = (BN - 1 - lcol).astype(jnp.float32)
        revr_ref[...] = (BM - 1 - lrow).astype(jnp.float32)
        for blk in range(NI):
            bs = bank_ref[blk * BM:(blk + 1) * BM, :]
            sq = bs * bs
            nb8 = lax.dot_general(jnp.ones((8, D), jnp.float32), sq,
                                  (((1,), (1,)), ((), ())),
                                  preferred_element_type=jnp.float32)
            nrow_ref[pl.ds(blk, 1), :, :] = nb8.reshape(1, 8, BN)
            na = jnp.sum(sq, axis=1, keepdims=True)
            napg_ref[pl.ds(blk, 1), :, :] = na.reshape(1, BM, 1)

    tp = jnp.maximum(s - 1, 0)
    i_p, jj_p, j_p = _coords(tp)

    u = gbuf[...]
    nb = nrow_ref[pl.ds(j_p, 1), :, :].reshape(8, BN)[0:1, :]
    colmask = lax.broadcasted_iota(jnp.int32, (BM, NI), 1) == i_p

    d = nb - u
    m = jnp.min(d, axis=1, keepdims=True)
    mxr = jnp.max(jnp.where(d == m, revc_ref[...], -1.0), axis=1,
                  keepdims=True)
    idx = (BN - 1) - mxr.astype(jnp.int32) + j_p * BN

    nacol = napg_ref[pl.ds(i_p, 1), :, :].reshape(BM, 1)
    dc = nacol - u
    mc = jnp.min(dc, axis=0, keepdims=True)
    mxc = jnp.max(jnp.where(dc == mc, revr_ref[...], -1.0), axis=0,
                  keepdims=True)
    idc = (BM - 1) - mxc.astype(jnp.int32) + i_p * BM

    i_c, jj_c, j_c = _coords(jnp.minimum(s, NT - 1))
    a = bank_ref[pl.ds(pl.multiple_of(i_c * BM, BM), BM), :]
    b2 = bank2_ref[pl.ds(pl.multiple_of(j_c * BN, BN), BN), :]
    page = jnp.where(jj_c == 0, 1, 0)
    gnew = lax.dot_general(a, b2, (((1,), (1,)), ((), ())),
                           preferred_element_type=jnp.float32)
    gnew = gnew + bias_ref[pl.ds(page, 1), :, :].reshape(BM, BN)

    @pl.when(s >= 1)
    def _merge_states():
        bv = rv_ref[...]
        bi = ri_ref[...]
        upd = colmask & ((m < bv) | ((m == bv) & (idx < bi)))
        rv_ref[...] = jnp.where(upd, m, bv)
        ri_ref[...] = jnp.where(upd, idx, bi)
        cbv = cv_ref[...]
        cbi = ci_ref[...]
        rowmask = lax.broadcasted_iota(jnp.int32, (NI, BN), 0) == j_p
        cupd = rowmask & ((mc < cbv) | ((mc == cbv) & (idc < cbi)))
        cv_ref[...] = jnp.where(cupd, mc, cbv)
        ci_ref[...] = jnp.where(cupd, idc, cbi)

    gbuf[...] = gnew


def _nearest_neighbor_halves(feature_bank):
    rv, ri, cv, ci = pl.pallas_call(
        _argmin_body,
        grid=(NT + 1,),
        in_specs=[pl.BlockSpec((N, D), lambda s: (0, 0))],
        out_specs=[
            pl.BlockSpec((BM, NI), lambda s: (0, 0)),
            pl.BlockSpec((BM, NI), lambda s: (0, 0)),
            pl.BlockSpec((NI, BN), lambda s: (0, 0)),
            pl.BlockSpec((NI, BN), lambda s: (0, 0)),
        ],
        out_shape=[
            jax.ShapeDtypeStruct((BM, NI), jnp.float32),
            jax.ShapeDtypeStruct((BM, NI), jnp.int32),
            jax.ShapeDtypeStruct((NI, BN), jnp.float32),
            jax.ShapeDtypeStruct((NI, BN), jnp.int32),
        ],
        scratch_shapes=[
            pltpu.VMEM((BM, BN), jnp.float32),
            pltpu.VMEM((NI, 8, BN), jnp.float32),
            pltpu.VMEM((NI, BM, 1), jnp.float32),
            pltpu.VMEM((N, D), jnp.float32),
            pltpu.VMEM((2, BM, BN), jnp.float32),
            pltpu.VMEM((BM, BN), jnp.float32),
            pltpu.VMEM((BM, BN), jnp.float32),
        ],
    )(feature_bank)
    return (rv.T.reshape(N), ri.T.reshape(N),
            cv.reshape(N), ci.reshape(N))


_NC = 2
_NS = 16
_NW = _NC * _NS
_CHUNK = N // _NW
_L = 16


def _sc_merge_and_count(rv, ri, cv, ci, label_bank):
    mesh = plsc.VectorSubcoreMesh(core_axis_name="c", subcore_axis_name="s")

    @functools.partial(
        pl.kernel,
        mesh=mesh,
        out_type=jax.ShapeDtypeStruct((_NW, _L), jnp.int32),
        scratch_types=[
            pltpu.VMEM((N,), jnp.int32),
            pltpu.VMEM((_CHUNK,), jnp.float32),
            pltpu.VMEM((_CHUNK,), jnp.int32),
            pltpu.VMEM((_CHUNK,), jnp.float32),
            pltpu.VMEM((_CHUNK,), jnp.int32),
            pltpu.VMEM((_CHUNK,), jnp.int32),
            pltpu.VMEM((_L,), jnp.int32),
        ],
        compiler_params=pltpu.CompilerParams(needs_layout_passes=False),
    )
    def k(rv_hbm, ri_hbm, cv_hbm, ci_hbm, label_hbm, out_hbm,
          table_v, rv_v, ri_v, cv_v, ci_v, own_v, acc_v):
        wid = lax.axis_index("s") * _NC + lax.axis_index("c")
        base = wid * _CHUNK
        pltpu.sync_copy(label_hbm, table_v)
        pltpu.sync_copy(rv_hbm.at[pl.ds(base, _CHUNK)], rv_v)
        pltpu.sync_copy(ri_hbm.at[pl.ds(base, _CHUNK)], ri_v)
        pltpu.sync_copy(cv_hbm.at[pl.ds(base, _CHUNK)], cv_v)
        pltpu.sync_copy(ci_hbm.at[pl.ds(base, _CHUNK)], ci_v)
        pltpu.sync_copy(label_hbm.at[pl.ds(base, _CHUNK)], own_v)
        acc = jnp.zeros((_L,), jnp.int32)
        for t in range(_CHUNK // _L):
            sl = pl.ds(t * _L, _L)
            rvv, riv = rv_v[sl], ri_v[sl]
            cvv, civ = cv_v[sl], ci_v[sl]
            own = own_v[sl]
            sel = (cvv < rvv) | ((cvv == rvv) & (civ < riv))
            pred = jnp.where(sel, civ, riv)
            g = plsc.load_gather(table_v, [pred])
            acc = acc + jnp.where(g == own, 1, 0).astype(jnp.int32)
        acc_v[...] = acc
        pltpu.sync_copy(acc_v, out_hbm.at[wid])

    return k(rv, ri, cv, ci, label_bank)


def kernel(feature_bank, label_bank):
    rv, ri, cv, ci = _nearest_neighbor_halves(feature_bank)
    counts = _sc_merge_and_count(rv, ri, cv, ci, label_bank)
    return jnp.sum(counts).astype(jnp.float32) / jnp.float32(N)
